# baseline (speedup 1.0000x reference)
import functools
import math

import jax
import jax.numpy as jnp
from jax.experimental import pallas as pl

N = 10000
E = 160000
B = 100
S = 4
P = 4
NMAX = 8
RC = 5.0
L9 = 9
Q = P * NMAX
C_F = L9 * Q

EB = 2000
AW = 256
N_PAD = 10496


def _edge_kernel(rvec_ref, aw_ref, sid_ref, c_ref):
    pid = pl.program_id(0)

    @pl.when(pid == 0)
    def _init():
        c_ref[...] = jnp.zeros_like(c_ref)

    rvec = rvec_ref[...]
    awp = aw_ref[...]
    sid = sid_ref[...]

    x = rvec[:, 0:1]
    y = rvec[:, 1:2]
    z = rvec[:, 2:3]
    r = jnp.sqrt(x * x + y * y + z * z + 1e-12)
    inv_r = 1.0 / r
    ux = x * inv_r
    uy = y * inv_r
    uz = z * inv_r

    fc = 0.5 * (jnp.cos(jnp.pi * jnp.minimum(r, RC) / RC) + 1.0)
    fc = fc * (r < RC).astype(jnp.float32)

    j_lane = jax.lax.broadcasted_iota(jnp.int32, (EB, NMAX), 1).astype(jnp.float32)
    mu = j_lane * (RC / (NMAX - 1))
    sigma = RC / NMAX
    dr = r - mu
    rb = jnp.exp(-(dr * dr) / (2.0 * sigma * sigma)) * fc

    y0 = jnp.full_like(r, 0.28209479)
    y1 = 0.48860251 * uy
    y2 = 0.48860251 * uz
    y3 = 0.48860251 * ux
    y4 = 1.09254843 * ux * uy
    y5 = 1.09254843 * uy * uz
    y6 = 0.31539157 * (3.0 * uz * uz - 1.0)
    y7 = 1.09254843 * ux * uz
    y8 = 0.54627421 * (ux * ux - uy * uy)
    ys = [y0, y1, y2, y3, y4, y5, y6, y7, y8]

    awrb = jnp.concatenate([awp[:, p:p + 1] * rb for p in range(P)], axis=1)
    feat = jnp.concatenate([yl * awrb for yl in ys], axis=1)

    lo = jnp.min(sid)
    hi = jnp.max(sid)
    w0 = lo // AW
    nw = hi // AW - w0 + 1
    lane = jax.lax.broadcasted_iota(jnp.int32, (EB, AW), 1)

    def body(w, carry):
        start = (w0 + w) * AW
        oh = (sid == start + lane).astype(jnp.float32)
        contrib = jax.lax.dot_general(
            oh, feat, (((0,), (0,)), ((), ())),
            preferred_element_type=jnp.float32)
        c_ref[pl.ds(start, AW), :] += contrib
        return carry

    jax.lax.fori_loop(0, nw, body, 0)


def _atom_kernel(c_ref, ohn_ref, ohs_ref, w1t_ref, w2t_ref, w3t_ref,
                 wpst_ref, wct_ref, acc_ref):
    pid = pl.program_id(0)

    @pl.when(pid == 0)
    def _init():
        acc_ref[...] = jnp.zeros_like(acc_ref)

    cb = c_ref[...]

    jrow = jax.lax.broadcasted_iota(jnp.int32, (Q, Q * Q), 1)
    qrow = jax.lax.broadcasted_iota(jnp.int32, (Q, Q * Q), 0)
    r_rep = (jrow // Q == qrow).astype(jnp.float32)
    r_til = (jrow % Q == qrow).astype(jnp.float32)

    groups = ((0, 1, 1.0), (1, 4, 1.0 / math.sqrt(3.0)),
              (4, 9, 1.0 / math.sqrt(5.0)))
    parts = []
    for l0, l1, scale in groups:
        accg = None
        for l in range(l0, l1):
            cl = cb[:, l * Q:(l + 1) * Q]
            a = jnp.dot(cl, r_rep, preferred_element_type=jnp.float32)
            b = jnp.dot(cl, r_til, preferred_element_type=jnp.float32)
            term = a * b
            accg = term if accg is None else accg + term
        parts.append(accg * scale)
    ps = jnp.concatenate(parts, axis=1)

    h1 = jnp.dot(ps, w1t_ref[...], preferred_element_type=jnp.float32)
    h1 = h1 * jax.nn.sigmoid(h1)
    h2 = jnp.dot(h1, w2t_ref[...], preferred_element_type=jnp.float32)
    h2 = h2 * jax.nn.sigmoid(h2)
    e_nn = jnp.dot(h2, w3t_ref[...], preferred_element_type=jnp.float32)
    e_ps = jnp.dot(ps, wpst_ref[...], preferred_element_type=jnp.float32)
    e_comp = jnp.dot(ohn_ref[...], wct_ref[...],
                     preferred_element_type=jnp.float32)
    e_atom = e_nn + e_ps + e_comp

    contrib = jax.lax.dot_general(
        e_atom, ohs_ref[...], (((0,), (0,)), ((), ())),
        preferred_element_type=jnp.float32)
    acc_ref[0:1, :] += contrib


@jax.jit
def kernel(positions, cells, numbers, edge_indices, edge_shifts, ptr,
           W_comp, pseudo_emb, W_pslin, W1, W2, W3):
    n = positions.shape[0]
    e = edge_indices.shape[1]
    struct_id = jnp.searchsorted(ptr, jnp.arange(n), side='right') - 1

    src = edge_indices[0]
    dst = edge_indices[1]
    order = jnp.arange(e, dtype=jnp.int32)
    srcs = src[order]
    dsts = dst[order]
    shifts_s = edge_shifts[order]
    cell_e = cells[struct_id[srcs]]
    rvec = (positions[dsts] - positions[srcs]
            + jnp.einsum('ek,ekl->el', shifts_s, cell_e))
    awp = pseudo_emb[numbers[dsts]]
    sid = (jnp.arange(e, dtype=jnp.int32) * n // e)[:, None]

    grid_a = e // EB
    c = pl.pallas_call(
        _edge_kernel,
        grid=(grid_a,),
        in_specs=[
            pl.BlockSpec((EB, 3), lambda i: (i, 0)),
            pl.BlockSpec((EB, 4), lambda i: (i, 0)),
            pl.BlockSpec((EB, 1), lambda i: (i, 0)),
        ],
        out_specs=pl.BlockSpec((N_PAD, C_F), lambda i: (0, 0)),
        out_shape=jax.ShapeDtypeStruct((N_PAD, C_F), jnp.float32),
    )(rvec, awp, sid)

    numbers_p = jnp.concatenate(
        [numbers.astype(jnp.int32), jnp.zeros((N_PAD - n,), jnp.int32)])
    struct_p = jnp.concatenate(
        [struct_id.astype(jnp.int32), jnp.full((N_PAD - n,), 127, jnp.int32)])
    ohn = jax.nn.one_hot(numbers_p, 8, dtype=jnp.float32)
    ohs = jax.nn.one_hot(struct_p, 128, dtype=jnp.float32)
    wct = jnp.zeros((8, 1), jnp.float32).at[:S, 0].set(W_comp[0])

    grid_b = N_PAD // AW
    acc = pl.pallas_call(
        _atom_kernel,
        grid=(grid_b,),
        in_specs=[
            pl.BlockSpec((AW, C_F), lambda i: (i, 0)),
            pl.BlockSpec((AW, 8), lambda i: (i, 0)),
            pl.BlockSpec((AW, 128), lambda i: (i, 0)),
            pl.BlockSpec(W1.T.shape, lambda i: (0, 0)),
            pl.BlockSpec(W2.T.shape, lambda i: (0, 0)),
            pl.BlockSpec(W3.T.shape, lambda i: (0, 0)),
            pl.BlockSpec(W_pslin.T.shape, lambda i: (0, 0)),
            pl.BlockSpec((8, 1), lambda i: (0, 0)),
        ],
        out_specs=pl.BlockSpec((8, 128), lambda i: (0, 0)),
        out_shape=jax.ShapeDtypeStruct((8, 128), jnp.float32),
    )(c, ohn, ohs, W1.T, W2.T, W3.T, W_pslin.T, wct)

    b = ptr.shape[0] - 1
    return acc[0, :b][:, None]


# speedup vs baseline: 1.7922x; 1.7922x over previous
import functools
import math

import jax
import jax.numpy as jnp
from jax.experimental import pallas as pl

N = 10000
E = 160000
B = 100
S = 4
P = 4
NMAX = 8
RC = 5.0
L9 = 9
Q = P * NMAX
C_F = L9 * Q

EB = 2000
AW = 256
N_PAD = 10496


def _edge_kernel(es_ref, rs_ref, rd_ref, sid_ref, c_ref):
    pid = pl.program_id(0)

    @pl.when(pid == 0)
    def _init():
        c_ref[...] = jnp.zeros_like(c_ref)

    es = es_ref[...]
    rs = rs_ref[...]
    rd = rd_ref[...]
    sid = sid_ref[...]
    awp = rd[:, 3:7]

    x = rd[:, 0:1] - rs[:, 0:1]
    y = rd[:, 1:2] - rs[:, 1:2]
    z = rd[:, 2:3] - rs[:, 2:3]
    s0 = es[:, 2:3]
    s1 = es[:, 3:4]
    s2 = es[:, 4:5]
    x = x + s0 * rs[:, 3:4] + s1 * rs[:, 6:7] + s2 * rs[:, 9:10]
    y = y + s0 * rs[:, 4:5] + s1 * rs[:, 7:8] + s2 * rs[:, 10:11]
    z = z + s0 * rs[:, 5:6] + s1 * rs[:, 8:9] + s2 * rs[:, 11:12]
    r = jnp.sqrt(x * x + y * y + z * z + 1e-12)
    inv_r = 1.0 / r
    ux = x * inv_r
    uy = y * inv_r
    uz = z * inv_r

    fc = 0.5 * (jnp.cos(jnp.pi * jnp.minimum(r, RC) / RC) + 1.0)
    fc = fc * (r < RC).astype(jnp.float32)

    j_lane = jax.lax.broadcasted_iota(jnp.int32, (EB, NMAX), 1).astype(jnp.float32)
    mu = j_lane * (RC / (NMAX - 1))
    sigma = RC / NMAX
    dr = r - mu
    rb = jnp.exp(-(dr * dr) / (2.0 * sigma * sigma)) * fc

    y0 = jnp.full_like(r, 0.28209479)
    y1 = 0.48860251 * uy
    y2 = 0.48860251 * uz
    y3 = 0.48860251 * ux
    y4 = 1.09254843 * ux * uy
    y5 = 1.09254843 * uy * uz
    y6 = 0.31539157 * (3.0 * uz * uz - 1.0)
    y7 = 1.09254843 * ux * uz
    y8 = 0.54627421 * (ux * ux - uy * uy)
    ys = [y0, y1, y2, y3, y4, y5, y6, y7, y8]

    awrb = jnp.concatenate([awp[:, p:p + 1] * rb for p in range(P)], axis=1)
    feat = jnp.concatenate([yl * awrb for yl in ys], axis=1)

    lo = jnp.min(sid)
    hi = jnp.max(sid)
    w0 = lo // AW
    nw = hi // AW - w0 + 1
    lane = jax.lax.broadcasted_iota(jnp.int32, (EB, AW), 1)

    def body(w, carry):
        start = (w0 + w) * AW
        oh = (sid == start + lane).astype(jnp.float32)
        contrib = jax.lax.dot_general(
            oh, feat, (((0,), (0,)), ((), ())),
            preferred_element_type=jnp.float32)
        c_ref[pl.ds(start, AW), :] += contrib
        return carry

    jax.lax.fori_loop(0, nw, body, 0)


def _atom_kernel(c_ref, ohn_ref, ohs_ref, w1t_ref, w2t_ref, w3t_ref,
                 wpst_ref, wct_ref, acc_ref):
    pid = pl.program_id(0)

    @pl.when(pid == 0)
    def _init():
        acc_ref[...] = jnp.zeros_like(acc_ref)

    cb = c_ref[...]

    jrow = jax.lax.broadcasted_iota(jnp.int32, (Q, Q * Q), 1)
    qrow = jax.lax.broadcasted_iota(jnp.int32, (Q, Q * Q), 0)
    r_rep = (jrow // Q == qrow).astype(jnp.float32)
    r_til = (jrow % Q == qrow).astype(jnp.float32)

    groups = ((0, 1, 1.0), (1, 4, 1.0 / math.sqrt(3.0)),
              (4, 9, 1.0 / math.sqrt(5.0)))
    parts = []
    for l0, l1, scale in groups:
        accg = None
        for l in range(l0, l1):
            cl = cb[:, l * Q:(l + 1) * Q]
            a = jnp.dot(cl, r_rep, preferred_element_type=jnp.float32)
            b = jnp.dot(cl, r_til, preferred_element_type=jnp.float32)
            term = a * b
            accg = term if accg is None else accg + term
        parts.append(accg * scale)
    ps = jnp.concatenate(parts, axis=1)

    h1 = jnp.dot(ps, w1t_ref[...], preferred_element_type=jnp.float32)
    h1 = h1 * jax.nn.sigmoid(h1)
    h2 = jnp.dot(h1, w2t_ref[...], preferred_element_type=jnp.float32)
    h2 = h2 * jax.nn.sigmoid(h2)
    e_nn = jnp.dot(h2, w3t_ref[...], preferred_element_type=jnp.float32)
    e_ps = jnp.dot(ps, wpst_ref[...], preferred_element_type=jnp.float32)
    e_comp = jnp.dot(ohn_ref[...], wct_ref[...],
                     preferred_element_type=jnp.float32)
    e_atom = e_nn + e_ps + e_comp

    contrib = jax.lax.dot_general(
        e_atom, ohs_ref[...], (((0,), (0,)), ((), ())),
        preferred_element_type=jnp.float32)
    acc_ref[0:1, :] += contrib


@jax.jit
def kernel(positions, cells, numbers, edge_indices, edge_shifts, ptr,
           W_comp, pseudo_emb, W_pslin, W1, W2, W3):
    n = positions.shape[0]
    e = edge_indices.shape[1]
    struct_id = jnp.searchsorted(ptr, jnp.arange(n), side='right') - 1

    src = edge_indices[0].astype(jnp.int32)
    dst = edge_indices[1].astype(jnp.int32)
    order = jnp.argsort(src)

    et = jnp.concatenate(
        [src.astype(jnp.float32)[:, None], dst.astype(jnp.float32)[:, None],
         edge_shifts], axis=1)
    et_s = et[order]
    srcs = et_s[:, 0].astype(jnp.int32)
    dsts = et_s[:, 1].astype(jnp.int32)
    sid = srcs[:, None]

    percell = cells.reshape(-1, 9)[struct_id]
    tab_src = jnp.concatenate([positions, percell], axis=1)
    tab_dst = jnp.concatenate(
        [positions, pseudo_emb[numbers]], axis=1)
    row_src = tab_src[srcs]
    row_dst = tab_dst[dsts]

    grid_a = e // EB
    c = pl.pallas_call(
        _edge_kernel,
        grid=(grid_a,),
        in_specs=[
            pl.BlockSpec((EB, 5), lambda i: (i, 0)),
            pl.BlockSpec((EB, 12), lambda i: (i, 0)),
            pl.BlockSpec((EB, 7), lambda i: (i, 0)),
            pl.BlockSpec((EB, 1), lambda i: (i, 0)),
        ],
        out_specs=pl.BlockSpec((N_PAD, C_F), lambda i: (0, 0)),
        out_shape=jax.ShapeDtypeStruct((N_PAD, C_F), jnp.float32),
    )(et_s, row_src, row_dst, sid)

    numbers_p = jnp.concatenate(
        [numbers.astype(jnp.int32), jnp.zeros((N_PAD - n,), jnp.int32)])
    struct_p = jnp.concatenate(
        [struct_id.astype(jnp.int32), jnp.full((N_PAD - n,), 127, jnp.int32)])
    ohn = jax.nn.one_hot(numbers_p, 8, dtype=jnp.float32)
    ohs = jax.nn.one_hot(struct_p, 128, dtype=jnp.float32)
    wct = jnp.zeros((8, 1), jnp.float32).at[:S, 0].set(W_comp[0])

    grid_b = N_PAD // AW
    acc = pl.pallas_call(
        _atom_kernel,
        grid=(grid_b,),
        in_specs=[
            pl.BlockSpec((AW, C_F), lambda i: (i, 0)),
            pl.BlockSpec((AW, 8), lambda i: (i, 0)),
            pl.BlockSpec((AW, 128), lambda i: (i, 0)),
            pl.BlockSpec(W1.T.shape, lambda i: (0, 0)),
            pl.BlockSpec(W2.T.shape, lambda i: (0, 0)),
            pl.BlockSpec(W3.T.shape, lambda i: (0, 0)),
            pl.BlockSpec(W_pslin.T.shape, lambda i: (0, 0)),
            pl.BlockSpec((8, 1), lambda i: (0, 0)),
        ],
        out_specs=pl.BlockSpec((8, 128), lambda i: (0, 0)),
        out_shape=jax.ShapeDtypeStruct((8, 128), jnp.float32),
    )(c, ohn, ohs, W1.T, W2.T, W3.T, W_pslin.T, wct)

    b = ptr.shape[0] - 1
    return acc[0, :b][:, None]
